# baseline (speedup 1.0000x reference)
import functools

import jax
import jax.numpy as jnp
import numpy as np
from jax import lax
from jax.experimental import pallas as pl
from jax.experimental.pallas import tpu as pltpu
from jax.experimental.pallas import tpu_sc as plsc

_VOCAB = 100000
_EMBED = 128
_DMODEL = 768
_MAXLEN = 2048
_EPS = 1e-5
_BLK = 1024
_NCHUNK = 2


def _sinusoidal_pe_np(max_len, d):
    pos = np.arange(max_len, dtype=np.float32)[:, None]
    div = np.exp(np.arange(0, d, 2, dtype=np.float32) * (-np.log(10000.0) / d))
    pe = np.zeros((max_len, d), dtype=np.float32)
    pe[:, 0::2] = np.sin(pos * div)
    pe[:, 1::2] = np.cos(pos * div)
    return pe


def _sc_gather(token_table, idx_2d, n_rows):
    info = plsc.get_sparse_core_info()
    nc, ns = info.num_cores, info.num_subcores
    nw = nc * ns
    rows_per_w = n_rows // nw

    mesh = plsc.VectorSubcoreMesh(core_axis_name="c", subcore_axis_name="s")

    @functools.partial(
        pl.kernel,
        mesh=mesh,
        out_type=jax.ShapeDtypeStruct((n_rows, _EMBED), jnp.float32),
        scratch_types=[
            pltpu.VMEM((1, rows_per_w), jnp.int32),
            pltpu.VMEM((rows_per_w, _EMBED), jnp.float32),
            pltpu.SemaphoreType.DMA,
        ],
    )
    def gather_kernel(table_hbm, idx_hbm, out_hbm, idx_v, rows_v, sem):
        wid = lax.axis_index("s") * nc + lax.axis_index("c")
        pltpu.sync_copy(idx_hbm.at[pl.ds(wid, 1)], idx_v)
        pltpu.async_copy(table_hbm.at[idx_v.at[0]], rows_v, sem).wait()
        pltpu.sync_copy(rows_v, out_hbm.at[pl.ds(wid * rows_per_w, rows_per_w)])

    return gather_kernel(token_table, idx_2d)


def _tc_body(s_len, g_ref, pe_ref, seg_ref, segtab_ref, w_ref, b_ref,
             gamma_ref, beta_ref, *rest):
    out_ref = rest[-1]
    j = pl.program_id(0)
    pe_off = (j % (s_len // _BLK)) * _BLK
    x = g_ref[...] + pe_ref[pl.ds(pe_off, _BLK), :]
    seg = seg_ref[...]
    for r in range(3):
        mask = jnp.where(seg == r, 1.0, 0.0)
        x = x + mask * segtab_ref[r, :][None, :]
    y = jnp.dot(x, w_ref[...], preferred_element_type=jnp.float32)
    y = y + b_ref[...]
    mu = jnp.mean(y, axis=-1, keepdims=True)
    d = y - mu
    var = jnp.mean(d * d, axis=-1, keepdims=True)
    yn = d * lax.rsqrt(var + _EPS)
    out_ref[...] = yn * gamma_ref[...] + beta_ref[...]


def _tc_fused_chunk(g, pe, seg_col, segtab_pad, W, b, gamma, beta, prev_out,
                    chunk, n_rows, s_len):
    blocks_per_chunk = (n_rows // _NCHUNK) // _BLK
    base = chunk * blocks_per_chunk

    in_specs = [
        pl.BlockSpec((_BLK, _EMBED), lambda j: (j, 0)),
        pl.BlockSpec((s_len, _EMBED), lambda j: (0, 0)),
        pl.BlockSpec((_BLK, 1), lambda j: (j + base, 0)),
        pl.BlockSpec((8, _EMBED), lambda j: (0, 0)),
        pl.BlockSpec((_EMBED, _DMODEL), lambda j: (0, 0)),
        pl.BlockSpec((1, _DMODEL), lambda j: (0, 0)),
        pl.BlockSpec((1, _DMODEL), lambda j: (0, 0)),
        pl.BlockSpec((1, _DMODEL), lambda j: (0, 0)),
    ]
    args = [g, pe, seg_col, segtab_pad, W, b, gamma, beta]
    aliases = {}
    if prev_out is not None:
        in_specs.append(pl.BlockSpec(memory_space=pl.ANY))
        args.append(prev_out)
        aliases = {8: 0}

    return pl.pallas_call(
        functools.partial(_tc_body, s_len),
        grid=(blocks_per_chunk,),
        in_specs=in_specs,
        out_specs=pl.BlockSpec((_BLK, _DMODEL), lambda j: (j + base, 0)),
        out_shape=jax.ShapeDtypeStruct((n_rows, _DMODEL), jnp.float32),
        input_output_aliases=aliases,
    )(*args)


def kernel(sequence, sequence_segment, token_table, seg_table, W, b, gamma, beta):
    bsz, s_len = sequence.shape
    n_rows = bsz * s_len
    rows_per_chunk = n_rows // _NCHUNK

    idx = jnp.reshape(sequence.astype(jnp.int32), (_NCHUNK, 32, rows_per_chunk // 32))
    gathered = [_sc_gather(token_table, idx[k], rows_per_chunk)
                for k in range(_NCHUNK)]

    pe = jnp.asarray(_sinusoidal_pe_np(_MAXLEN, _EMBED)[:s_len])
    seg_col = jnp.reshape(sequence_segment.astype(jnp.int32), (n_rows, 1))
    segtab_pad = jnp.zeros((8, _EMBED), jnp.float32).at[:3].set(seg_table)
    b2 = jnp.reshape(b, (1, _DMODEL))
    gamma2 = jnp.reshape(gamma, (1, _DMODEL))
    beta2 = jnp.reshape(beta, (1, _DMODEL))

    out = None
    for k in range(_NCHUNK):
        out = _tc_fused_chunk(gathered[k], pe, seg_col, segtab_pad, W, b2,
                              gamma2, beta2, out, k, n_rows, s_len)
    return jnp.reshape(out, (bsz, s_len, _DMODEL))


# speedup vs baseline: 1.0387x; 1.0387x over previous
import functools

import jax
import jax.numpy as jnp
import numpy as np
from jax import lax
from jax.experimental import pallas as pl
from jax.experimental.pallas import tpu as pltpu
from jax.experimental.pallas import tpu_sc as plsc

_VOCAB = 100000
_EMBED = 128
_DMODEL = 768
_MAXLEN = 2048
_EPS = 1e-5
_BLK = 1024
_NCHUNK = 2


def _sinusoidal_pe_np(max_len, d):
    pos = np.arange(max_len, dtype=np.float32)[:, None]
    div = np.exp(np.arange(0, d, 2, dtype=np.float32) * (-np.log(10000.0) / d))
    pe = np.zeros((max_len, d), dtype=np.float32)
    pe[:, 0::2] = np.sin(pos * div)
    pe[:, 1::2] = np.cos(pos * div)
    return pe


def _sc_gather(token_table, idx_2d, chunk, n_rows):
    info = plsc.get_sparse_core_info()
    nc, ns = info.num_cores, info.num_subcores
    nw = nc * ns
    rows_per_w = n_rows // nw

    mesh = plsc.VectorSubcoreMesh(core_axis_name="c", subcore_axis_name="s")

    @functools.partial(
        pl.kernel,
        mesh=mesh,
        out_type=jax.ShapeDtypeStruct((n_rows, _EMBED), jnp.float32),
        scratch_types=[
            pltpu.VMEM((1, rows_per_w), jnp.int32),
            pltpu.VMEM((rows_per_w, _EMBED), jnp.float32),
            pltpu.SemaphoreType.DMA,
        ],
    )
    def gather_kernel(table_hbm, idx_hbm, out_hbm, idx_v, rows_v, sem):
        wid = lax.axis_index("s") * nc + lax.axis_index("c")
        pltpu.sync_copy(idx_hbm.at[pl.ds(chunk * nw + wid, 1)], idx_v)
        pltpu.async_copy(table_hbm.at[idx_v.at[0]], rows_v, sem).wait()
        pltpu.sync_copy(rows_v, out_hbm.at[pl.ds(wid * rows_per_w, rows_per_w)])

    return gather_kernel(token_table, idx_2d)


def _tc_body(s_len, g_ref, pe_ref, seg_ref, segtab_ref, w_ref, bgb_ref, *rest):
    out_ref = rest[-1]
    j = pl.program_id(0)
    pe_off = (j % (s_len // _BLK)) * _BLK
    x = g_ref[...] + pe_ref[pl.ds(pe_off, _BLK), :]
    seg = seg_ref[...].astype(jnp.int32)
    for r in range(3):
        mask = jnp.where(seg == r, 1.0, 0.0)
        x = x + mask * segtab_ref[r, :][None, :]
    y = jnp.dot(x, w_ref[...], preferred_element_type=jnp.float32)
    y = y + bgb_ref[0, :][None, :]
    mu = jnp.mean(y, axis=-1, keepdims=True)
    d = y - mu
    var = jnp.mean(d * d, axis=-1, keepdims=True)
    yn = d * lax.rsqrt(var + _EPS)
    out_ref[...] = yn * bgb_ref[1, :][None, :] + bgb_ref[2, :][None, :]


def _tc_fused_chunk(g, pe, seg_col, segtab_pad, W, bgb, prev_out,
                    chunk, n_rows, s_len):
    blocks_per_chunk = (n_rows // _NCHUNK) // _BLK
    base = chunk * blocks_per_chunk

    in_specs = [
        pl.BlockSpec((_BLK, _EMBED), lambda j: (j, 0)),
        pl.BlockSpec((s_len, _EMBED), lambda j: (0, 0)),
        pl.BlockSpec((_BLK, 1), lambda j: (j + base, 0)),
        pl.BlockSpec((8, _EMBED), lambda j: (0, 0)),
        pl.BlockSpec((_EMBED, _DMODEL), lambda j: (0, 0)),
        pl.BlockSpec((3, _DMODEL), lambda j: (0, 0)),
    ]
    args = [g, pe, seg_col, segtab_pad, W, bgb]
    aliases = {}
    if prev_out is not None:
        in_specs.append(pl.BlockSpec(memory_space=pl.ANY))
        args.append(prev_out)
        aliases = {6: 0}

    return pl.pallas_call(
        functools.partial(_tc_body, s_len),
        grid=(blocks_per_chunk,),
        in_specs=in_specs,
        out_specs=pl.BlockSpec((_BLK, _DMODEL), lambda j: (j + base, 0)),
        out_shape=jax.ShapeDtypeStruct((n_rows, _DMODEL), jnp.float32),
        input_output_aliases=aliases,
    )(*args)


def kernel(sequence, sequence_segment, token_table, seg_table, W, b, gamma, beta):
    bsz, s_len = sequence.shape
    n_rows = bsz * s_len
    rows_per_chunk = n_rows // _NCHUNK

    idx = jnp.reshape(sequence.astype(jnp.int32), (n_rows // 128, 128))
    gathered = [_sc_gather(token_table, idx, k, rows_per_chunk)
                for k in range(_NCHUNK)]

    pe = jnp.asarray(_sinusoidal_pe_np(_MAXLEN, _EMBED)[:s_len])
    seg_col = jnp.reshape(sequence_segment.astype(jnp.int8), (n_rows, 1))
    segtab_pad = jnp.zeros((8, _EMBED), jnp.float32).at[:3].set(seg_table)
    bgb = jnp.stack([b, gamma, beta])

    out = None
    for k in range(_NCHUNK):
        out = _tc_fused_chunk(gathered[k], pe, seg_col, segtab_pad, W, bgb,
                              out, k, n_rows, s_len)
    return jnp.reshape(out, (bsz, s_len, _DMODEL))


# speedup vs baseline: 1.1126x; 1.0711x over previous
import functools

import jax
import jax.numpy as jnp
import numpy as np
from jax import lax
from jax.experimental import pallas as pl
from jax.experimental.pallas import tpu as pltpu
from jax.experimental.pallas import tpu_sc as plsc

_VOCAB = 100000
_EMBED = 128
_DMODEL = 768
_MAXLEN = 2048
_EPS = 1e-5
_BLK = 1024
_NCHUNK = 1


def _sinusoidal_pe_np(max_len, d):
    pos = np.arange(max_len, dtype=np.float32)[:, None]
    div = np.exp(np.arange(0, d, 2, dtype=np.float32) * (-np.log(10000.0) / d))
    pe = np.zeros((max_len, d), dtype=np.float32)
    pe[:, 0::2] = np.sin(pos * div)
    pe[:, 1::2] = np.cos(pos * div)
    return pe


def _sc_gather(token_table, idx_2d, chunk, n_rows):
    info = plsc.get_sparse_core_info()
    nc, ns = info.num_cores, info.num_subcores
    nw = nc * ns
    rows_per_w = n_rows // nw
    sub = rows_per_w // 128

    mesh = plsc.VectorSubcoreMesh(core_axis_name="c", subcore_axis_name="s")

    @functools.partial(
        pl.kernel,
        mesh=mesh,
        out_type=jax.ShapeDtypeStruct((n_rows, _EMBED), jnp.float32),
        scratch_types=[
            pltpu.VMEM((sub, 128), jnp.int32),
            pltpu.VMEM((rows_per_w, _EMBED), jnp.float32),
            pltpu.SemaphoreType.DMA,
        ],
    )
    def gather_kernel(table_hbm, idx_hbm, out_hbm, idx_v, rows_v, sem):
        wid = lax.axis_index("s") * nc + lax.axis_index("c")
        pltpu.sync_copy(idx_hbm.at[pl.ds((chunk * nw + wid) * sub, sub)], idx_v)
        copies = [
            pltpu.async_copy(table_hbm.at[idx_v.at[j]],
                             rows_v.at[pl.ds(j * 128, 128)], sem)
            for j in range(sub)
        ]
        for c in copies:
            c.wait()
        pltpu.sync_copy(rows_v, out_hbm.at[pl.ds(wid * rows_per_w, rows_per_w)])

    return gather_kernel(token_table, idx_2d)


def _tc_body(s_len, g_ref, pe_ref, seg_ref, segtab_ref, w_ref, bgb_ref, *rest):
    out_ref = rest[-1]
    j = pl.program_id(0)
    pe_off = (j % (s_len // _BLK)) * _BLK
    x = g_ref[...] + pe_ref[pl.ds(pe_off, _BLK), :]
    seg = seg_ref[...].astype(jnp.int32)
    for r in range(3):
        mask = jnp.where(seg == r, 1.0, 0.0)
        x = x + mask * segtab_ref[r, :][None, :]
    y = jnp.dot(x, w_ref[...], preferred_element_type=jnp.float32)
    y = y + bgb_ref[0, :][None, :]
    mu = jnp.mean(y, axis=-1, keepdims=True)
    d = y - mu
    var = jnp.mean(d * d, axis=-1, keepdims=True)
    yn = d * lax.rsqrt(var + _EPS)
    out_ref[...] = yn * bgb_ref[1, :][None, :] + bgb_ref[2, :][None, :]


def _tc_fused_chunk(g, pe, seg_col, segtab_pad, W, bgb, prev_out,
                    chunk, n_rows, s_len):
    blocks_per_chunk = (n_rows // _NCHUNK) // _BLK
    base = chunk * blocks_per_chunk

    in_specs = [
        pl.BlockSpec((_BLK, _EMBED), lambda j: (j, 0)),
        pl.BlockSpec((s_len, _EMBED), lambda j: (0, 0)),
        pl.BlockSpec((_BLK, 1), lambda j: (j + base, 0)),
        pl.BlockSpec((8, _EMBED), lambda j: (0, 0)),
        pl.BlockSpec((_EMBED, _DMODEL), lambda j: (0, 0)),
        pl.BlockSpec((3, _DMODEL), lambda j: (0, 0)),
    ]
    args = [g, pe, seg_col, segtab_pad, W, bgb]
    aliases = {}
    if prev_out is not None:
        in_specs.append(pl.BlockSpec(memory_space=pl.ANY))
        args.append(prev_out)
        aliases = {6: 0}

    return pl.pallas_call(
        functools.partial(_tc_body, s_len),
        grid=(blocks_per_chunk,),
        in_specs=in_specs,
        out_specs=pl.BlockSpec((_BLK, _DMODEL), lambda j: (j + base, 0)),
        out_shape=jax.ShapeDtypeStruct((n_rows, _DMODEL), jnp.float32),
        input_output_aliases=aliases,
    )(*args)


def kernel(sequence, sequence_segment, token_table, seg_table, W, b, gamma, beta):
    bsz, s_len = sequence.shape
    n_rows = bsz * s_len
    rows_per_chunk = n_rows // _NCHUNK

    idx = jnp.reshape(sequence.astype(jnp.int32), (n_rows // 128, 128))
    gathered = [_sc_gather(token_table, idx, k, rows_per_chunk)
                for k in range(_NCHUNK)]

    pe = jnp.asarray(_sinusoidal_pe_np(_MAXLEN, _EMBED)[:s_len])
    seg_col = jnp.reshape(sequence_segment.astype(jnp.int8), (n_rows, 1))
    segtab_pad = jnp.zeros((8, _EMBED), jnp.float32).at[:3].set(seg_table)
    bgb = jnp.stack([b, gamma, beta])

    out = None
    for k in range(_NCHUNK):
        out = _tc_fused_chunk(gathered[k], pe, seg_col, segtab_pad, W, bgb,
                              out, k, n_rows, s_len)
    return jnp.reshape(out, (bsz, s_len, _DMODEL))


# speedup vs baseline: 1.1390x; 1.0238x over previous
import functools

import jax
import jax.numpy as jnp
import numpy as np
from jax import lax
from jax.experimental import pallas as pl
from jax.experimental.pallas import tpu as pltpu
from jax.experimental.pallas import tpu_sc as plsc

_VOCAB = 100000
_EMBED = 128
_DMODEL = 768
_MAXLEN = 2048
_EPS = 1e-5
_BLK = 2048
_NCHUNK = 1


def _sinusoidal_pe_np(max_len, d):
    pos = np.arange(max_len, dtype=np.float32)[:, None]
    div = np.exp(np.arange(0, d, 2, dtype=np.float32) * (-np.log(10000.0) / d))
    pe = np.zeros((max_len, d), dtype=np.float32)
    pe[:, 0::2] = np.sin(pos * div)
    pe[:, 1::2] = np.cos(pos * div)
    return pe


def _sc_gather(token_table, idx_2d, chunk, n_rows):
    info = plsc.get_sparse_core_info()
    nc, ns = info.num_cores, info.num_subcores
    nw = nc * ns
    rows_per_w = n_rows // nw
    sub = rows_per_w // 128

    mesh = plsc.VectorSubcoreMesh(core_axis_name="c", subcore_axis_name="s")

    @functools.partial(
        pl.kernel,
        mesh=mesh,
        out_type=jax.ShapeDtypeStruct((n_rows, _EMBED), jnp.float32),
        scratch_types=[
            pltpu.VMEM((sub, 128), jnp.int32),
            pltpu.VMEM((rows_per_w, _EMBED), jnp.float32),
            pltpu.SemaphoreType.DMA,
        ],
    )
    def gather_kernel(table_hbm, idx_hbm, out_hbm, idx_v, rows_v, sem):
        wid = lax.axis_index("s") * nc + lax.axis_index("c")
        pltpu.sync_copy(idx_hbm.at[pl.ds((chunk * nw + wid) * sub, sub)], idx_v)
        copies = [
            pltpu.async_copy(table_hbm.at[idx_v.at[j]],
                             rows_v.at[pl.ds(j * 128, 128)], sem)
            for j in range(sub)
        ]
        for c in copies:
            c.wait()
        pltpu.sync_copy(rows_v, out_hbm.at[pl.ds(wid * rows_per_w, rows_per_w)])

    return gather_kernel(token_table, idx_2d)


def _tc_body(s_len, g_ref, pe_ref, seg_ref, segtab_ref, w_ref, bgb_ref, *rest):
    out_ref = rest[-1]
    j = pl.program_id(0)
    pe_off = (j % (s_len // _BLK)) * _BLK
    x = g_ref[...] + pe_ref[pl.ds(pe_off, _BLK), :]
    seg = seg_ref[...].astype(jnp.int32)
    for r in range(3):
        mask = jnp.where(seg == r, 1.0, 0.0)
        x = x + mask * segtab_ref[r, :][None, :]
    y = jnp.dot(x, w_ref[...], preferred_element_type=jnp.float32)
    y = y + bgb_ref[0, :][None, :]
    mu = jnp.mean(y, axis=-1, keepdims=True)
    d = y - mu
    var = jnp.mean(d * d, axis=-1, keepdims=True)
    yn = d * lax.rsqrt(var + _EPS)
    out_ref[...] = yn * bgb_ref[1, :][None, :] + bgb_ref[2, :][None, :]


def _tc_fused_chunk(g, pe, seg_col, segtab_pad, W, bgb, prev_out,
                    chunk, n_rows, s_len):
    blocks_per_chunk = (n_rows // _NCHUNK) // _BLK
    base = chunk * blocks_per_chunk

    in_specs = [
        pl.BlockSpec((_BLK, _EMBED), lambda j: (j, 0)),
        pl.BlockSpec((s_len, _EMBED), lambda j: (0, 0)),
        pl.BlockSpec((_BLK, 1), lambda j: (j + base, 0)),
        pl.BlockSpec((8, _EMBED), lambda j: (0, 0)),
        pl.BlockSpec((_EMBED, _DMODEL), lambda j: (0, 0)),
        pl.BlockSpec((3, _DMODEL), lambda j: (0, 0)),
    ]
    args = [g, pe, seg_col, segtab_pad, W, bgb]
    aliases = {}
    if prev_out is not None:
        in_specs.append(pl.BlockSpec(memory_space=pl.ANY))
        args.append(prev_out)
        aliases = {6: 0}

    return pl.pallas_call(
        functools.partial(_tc_body, s_len),
        grid=(blocks_per_chunk,),
        in_specs=in_specs,
        out_specs=pl.BlockSpec((_BLK, _DMODEL), lambda j: (j + base, 0)),
        out_shape=jax.ShapeDtypeStruct((n_rows, _DMODEL), jnp.float32),
        input_output_aliases=aliases,
    )(*args)


def kernel(sequence, sequence_segment, token_table, seg_table, W, b, gamma, beta):
    bsz, s_len = sequence.shape
    n_rows = bsz * s_len
    rows_per_chunk = n_rows // _NCHUNK

    idx = jnp.reshape(sequence.astype(jnp.int32), (n_rows // 128, 128))
    gathered = [_sc_gather(token_table, idx, k, rows_per_chunk)
                for k in range(_NCHUNK)]

    pe = jnp.asarray(_sinusoidal_pe_np(_MAXLEN, _EMBED)[:s_len])
    seg_col = jnp.reshape(sequence_segment.astype(jnp.int8), (n_rows, 1))
    segtab_pad = jnp.zeros((8, _EMBED), jnp.float32).at[:3].set(seg_table)
    bgb = jnp.stack([b, gamma, beta])

    out = None
    for k in range(_NCHUNK):
        out = _tc_fused_chunk(gathered[k], pe, seg_col, segtab_pad, W, bgb,
                              out, k, n_rows, s_len)
    return jnp.reshape(out, (bsz, s_len, _DMODEL))
